# baseline (speedup 1.0000x reference)
import functools

import jax
import jax.numpy as jnp
from jax import lax
from jax.experimental import pallas as pl
from jax.experimental.pallas import tpu as pltpu
from jax.experimental.pallas import tpu_sc as plsc

B, T, DIM = 4096, 200, 32
NC, NS = 2, 16
NW = NC * NS
ROWS_PER_W = B // NW
RCHUNK = 8
N_CHUNKS = ROWS_PER_W // RCHUNK

_mesh = plsc.VectorSubcoreMesh(core_axis_name="c", subcore_axis_name="s")


@functools.partial(
    pl.kernel,
    mesh=_mesh,
    out_type=jax.ShapeDtypeStruct((B, T, DIM), jnp.float32),
    scratch_types=[
        pltpu.VMEM((ROWS_PER_W, T), jnp.int32),
        pltpu.VMEM((2, RCHUNK, T, DIM), jnp.float32),
        pltpu.SemaphoreType.DMA((2,)),
        pltpu.SemaphoreType.DMA((2,)),
    ],
    compiler_params=pltpu.CompilerParams(use_tc_tiling_on_sc=False),
)
def _emb_lookup(idx_hbm, table_hbm, out_hbm, idx_v, rows_v, gsem, osem):
    wid = lax.axis_index("s") * NC + lax.axis_index("c")
    b0 = wid * ROWS_PER_W

    pltpu.sync_copy(idx_hbm.at[pl.ds(b0, ROWS_PER_W)], idx_v)

    def gather(i):
        for k in range(RCHUNK):
            pltpu.async_copy(
                table_hbm.at[idx_v.at[i * RCHUNK + k]],
                rows_v.at[i % 2].at[k],
                gsem.at[i % 2],
            )

    def gather_wait(i):
        for k in range(RCHUNK):
            pltpu.make_async_copy(
                table_hbm.at[idx_v.at[i * RCHUNK + k]],
                rows_v.at[i % 2].at[k],
                gsem.at[i % 2],
            ).wait()

    def writeback(i):
        pltpu.async_copy(
            rows_v.at[i % 2],
            out_hbm.at[pl.ds(b0 + i * RCHUNK, RCHUNK)],
            osem.at[i % 2],
        )

    def writeback_wait(i):
        pltpu.make_async_copy(
            rows_v.at[i % 2],
            out_hbm.at[pl.ds(b0 + i * RCHUNK, RCHUNK)],
            osem.at[i % 2],
        ).wait()

    gather(0)
    for i in range(N_CHUNKS):
        gather_wait(i)
        if i + 1 < N_CHUNKS:
            if i >= 1:
                writeback_wait(i - 1)
            gather(i + 1)
        writeback(i)

    writeback_wait(N_CHUNKS - 2)
    writeback_wait(N_CHUNKS - 1)


def kernel(input, weight):
    return _emb_lookup(input.astype(jnp.int32), weight)


# speedup vs baseline: 1.2256x; 1.2256x over previous
import functools

import jax
import jax.numpy as jnp
from jax import lax
from jax.experimental import pallas as pl
from jax.experimental.pallas import tpu as pltpu
from jax.experimental.pallas import tpu_sc as plsc

V = 1000000
DIM = 32
B_TOTAL = 4096 * 200


CBLK = 4608
NSTEP = 217
RBLK = CBLK // 4
V_MAIN = NSTEP * CBLK
TROWS = V_MAIN // 4 + 32


def _fmt_body(wt_hbm, tail_hbm, out_hbm, xin, yv, isem, osem, tsem):
    i = pl.program_id(0)
    b = lax.rem(i, 2)
    nb = lax.rem(i + 1, 2)

    def in_cp(step, buf):
        return pltpu.make_async_copy(
            wt_hbm.at[:, pl.ds(step * CBLK, CBLK)], xin.at[buf], isem.at[buf]
        )

    def out_cp(step, buf):
        return pltpu.make_async_copy(
            yv.at[buf],
            out_hbm.at[pl.ds(step * RBLK, RBLK)],
            osem.at[buf],
        )

    @pl.when(i == 0)
    def _():
        in_cp(0, 0).start()

    @pl.when(i + 1 < NSTEP)
    def _():
        in_cp(i + 1, nb).start()

    in_cp(i, b).wait()

    @pl.when(i >= 2)
    def _():
        out_cp(i - 2, b).wait()

    x = xin[b]
    for k in range(4):
        yv[b, :, k * DIM:(k + 1) * DIM] = lax.transpose(
            x[:, k * RBLK:(k + 1) * RBLK], (1, 0)
        )
    out_cp(i, b).start()

    @pl.when(i == NSTEP - 1)
    def _():
        pltpu.make_async_copy(
            tail_hbm, out_hbm.at[pl.ds(NSTEP * RBLK, 16)], tsem
        ).start()
        out_cp(NSTEP - 2, nb).wait()
        out_cp(NSTEP - 1, b).wait()
        pltpu.make_async_copy(
            tail_hbm, out_hbm.at[pl.ds(NSTEP * RBLK, 16)], tsem
        ).wait()


_fmt = pl.pallas_call(
    _fmt_body,
    grid=(NSTEP,),
    in_specs=[
        pl.BlockSpec(memory_space=pl.ANY),
        pl.BlockSpec(memory_space=pl.ANY),
    ],
    out_specs=pl.BlockSpec(memory_space=pl.ANY),
    out_shape=jax.ShapeDtypeStruct((TROWS, 128), jnp.float32),
    scratch_shapes=[
        pltpu.VMEM((2, DIM, CBLK), jnp.float32),
        pltpu.VMEM((2, RBLK, 128), jnp.float32),
        pltpu.SemaphoreType.DMA((2,)),
        pltpu.SemaphoreType.DMA((2,)),
        pltpu.SemaphoreType.DMA,
    ],
)


NC, NS = 2, 16
NW = NC * NS
B_PER_W = B_TOTAL // NW
CHUNK = 1600
N_CHUNKS = B_PER_W // CHUNK
N_VREG = B_PER_W // 16

_mesh = plsc.VectorSubcoreMesh(core_axis_name="c", subcore_axis_name="s")


@functools.partial(
    pl.kernel,
    mesh=_mesh,
    out_type=jax.ShapeDtypeStruct((B_TOTAL, DIM), jnp.float32),
    scratch_types=[
        pltpu.VMEM((B_PER_W,), jnp.int32),
        pltpu.VMEM((2, CHUNK, DIM), jnp.float32),
        pltpu.SemaphoreType.DMA((2,)),
        pltpu.SemaphoreType.DMA((2,)),
    ],
    compiler_params=pltpu.CompilerParams(use_tc_tiling_on_sc=False),
)
def _emb_lookup(idx_hbm, table_hbm, out_hbm, idx_v, rows_v, gsem, osem):
    wid = lax.axis_index("s") * NC + lax.axis_index("c")
    base = wid * B_PER_W

    pltpu.sync_copy(idx_hbm.at[pl.ds(base, B_PER_W)], idx_v)

    def remap(v, _):
        ids = idx_v[pl.ds(v * 16, 16)]
        blk = ((ids >> 9) * 7282) >> 16
        m = ids - blk * CBLK
        k = ((m >> 7) * 7282) >> 16
        p = m - k * RBLK
        g = (blk * RBLK + p) * 4 + k
        idx_v[pl.ds(v * 16, 16)] = jnp.where(ids >= V_MAIN, ids, g)
        return _

    lax.fori_loop(0, N_VREG, remap, 0)

    def gat_cp(i):
        return pltpu.make_async_copy(
            table_hbm.at[idx_v.at[pl.ds(i * CHUNK, CHUNK)]],
            rows_v.at[i % 2],
            gsem.at[i % 2],
        )

    def out_cp(i):
        return pltpu.make_async_copy(
            rows_v.at[i % 2],
            out_hbm.at[pl.ds(base + i * CHUNK, CHUNK)],
            osem.at[i % 2],
        )

    gat_cp(0).start()
    for i in range(N_CHUNKS):
        gat_cp(i).wait()
        if i + 1 < N_CHUNKS:
            if i >= 1:
                out_cp(i - 1).wait()
            gat_cp(i + 1).start()
        out_cp(i).start()

    out_cp(N_CHUNKS - 2).wait()
    out_cp(N_CHUNKS - 1).wait()


def kernel(input, weight):
    idx = input.reshape(-1).astype(jnp.int32)
    tail = lax.slice(weight, (V_MAIN, 0), (V, DIM)).reshape(16, 128)
    table = _fmt(weight.T, tail).reshape(4 * TROWS, DIM)
    out = _emb_lookup(idx, table)
    return out.reshape(input.shape + (DIM,))
